# baseline (speedup 1.0000x reference)
import functools

import jax
import jax.numpy as jnp
from jax import lax
from jax.experimental import pallas as pl
from jax.experimental.pallas import tpu as pltpu
from jax.experimental.pallas import tpu_sc as plsc

D_MODEL = 64
SCALE = 8.0
NW = 32
B_TOTAL = 4096
T_TOTAL = 200
BW = B_TOTAL // NW
NSLOT = 2

_mesh = plsc.VectorSubcoreMesh(core_axis_name="c", subcore_axis_name="s")


@functools.partial(
    pl.kernel,
    mesh=_mesh,
    out_type=jax.ShapeDtypeStruct((T_TOTAL, D_MODEL, B_TOTAL), jnp.float32),
    scratch_types=[
        pltpu.VMEM((T_TOTAL, BW), jnp.int32),
        pltpu.VMEM((T_TOTAL, BW), jnp.int32),
        pltpu.VMEM((NSLOT, BW, 128), jnp.float32),
        pltpu.VMEM((NSLOT, D_MODEL, BW), jnp.float32),
        pltpu.SemaphoreType.DMA((NSLOT,)),
        pltpu.SemaphoreType.DMA((NSLOT,)),
    ],
    compiler_params=pltpu.CompilerParams(
        use_tc_tiling_on_sc=False, needs_layout_passes=False
    ),
)
def _emb_lookup(xt_hbm, lut2_hbm, out_hbm, idx_v, idx2_v, rows_v, tbuf_v,
                gsem, wsem):
    c = lax.axis_index("c")
    s = lax.axis_index("s")
    wid = s * 2 + c
    b0 = wid * BW

    pltpu.sync_copy(xt_hbm.at[:, pl.ds(b0, BW)], idx_v)

    @plsc.parallel_loop(0, T_TOTAL, unroll=2)
    def _(t):
        for j in range(BW // 16):
            sl = pl.ds(j * 16, 16)
            idx2_v[t, sl] = idx_v[t, sl] >> 1

    def fire_gather(slot, t):
        pltpu.async_copy(
            lut2_hbm.at[idx2_v.at[t]], rows_v.at[slot], gsem.at[slot]
        )

    def drain_gather(slot, t):
        pltpu.make_async_copy(
            lut2_hbm.at[idx2_v.at[t]], rows_v.at[slot], gsem.at[slot]
        ).wait()

    def fire_wb(slot, t):
        pltpu.async_copy(
            tbuf_v.at[slot], out_hbm.at[t, :, pl.ds(b0, BW)], wsem.at[slot]
        )

    def drain_wb(slot, t):
        pltpu.make_async_copy(
            tbuf_v.at[slot], out_hbm.at[t, :, pl.ds(b0, BW)], wsem.at[slot]
        ).wait()

    lanes = lax.iota(jnp.int32, 16)

    def transpose_scale(slot, t):
        @plsc.parallel_loop(0, BW // 16)
        def _(lg):
            bvec = lg * 16 + lanes
            par = (idx_v[t, pl.ds(lg * 16, 16)] & 1) * D_MODEL
            for d in range(D_MODEL):
                v = plsc.load_gather(rows_v.at[slot], [bvec, par + d])
                tbuf_v[slot, d, pl.ds(lg * 16, 16)] = v * SCALE

    fire_gather(0, 0)
    fire_gather(1, 1)

    def outer(i, carry):
        for b in range(NSLOT):
            t = i * NSLOT + b

            @pl.when(t >= NSLOT)
            def _():
                drain_wb(b, t - NSLOT)

            drain_gather(b, t)
            transpose_scale(b, t)

            @pl.when(t + NSLOT < T_TOTAL)
            def _():
                fire_gather(b, t + NSLOT)

            fire_wb(b, t)
        return carry

    lax.fori_loop(0, T_TOTAL // NSLOT, outer, 0)

    for b in range(NSLOT):
        drain_wb(b, T_TOTAL - NSLOT + b)


def kernel(x, lut):
    xt = x.T.astype(jnp.int32)
    lut2 = lut.reshape(500000, 128)
    out = _emb_lookup(xt, lut2)
    return jnp.transpose(out, (2, 0, 1))
